# baseline (speedup 1.0000x reference)
import numpy as np
import jax
import jax.numpy as jnp
from jax.experimental import pallas as pl
from jax.experimental.pallas import tpu as pltpu

_UP = 4
_TC = 10


def _resize_mat(n_in: int, n_out: int) -> np.ndarray:
    scale = n_out / n_in
    sample = (np.arange(n_out) + 0.5) / scale - 0.5
    dist = np.abs(sample[None, :] - np.arange(n_in)[:, None])
    w = np.maximum(0.0, 1.0 - dist)
    w = w / w.sum(axis=0, keepdims=True)
    return w.astype(np.float32)


def _fusion_body(slr0, slr1, shr0, shr1, lr_doy_s, hr_doy_s, tgt_s,
                 lr0, lr1, hr0, hr1, a_ref, at_ref, out_lr, out_hr):
    b = pl.program_id(0)
    t = pl.program_id(1)
    tc = t % _TC
    tval = tgt_s[t].astype(jnp.float32)

    def lerp_weight(doy_s, i0, i1):
        d0 = doy_s[b, i0].astype(jnp.float32)
        d1 = doy_s[b, i1].astype(jnp.float32)
        denom = jnp.where(d1 - d0 == 0.0, 1.0, d1 - d0)
        return jnp.clip((tval - d0) / denom, 0.0, 1.0)

    w_hr = lerp_weight(hr_doy_s, shr0[b, t], shr1[b, t])
    out_hr[0, tc] = hr0[0, 0] * (1.0 - w_hr) + hr1[0, 0] * w_hr

    w_lr = lerp_weight(lr_doy_s, slr0[b, t], slr1[b, t])
    x = lr0[0, 0] * (1.0 - w_lr) + lr1[0, 0] * w_lr
    a = a_ref[...]
    at = at_ref[...]
    for c in range(x.shape[0]):
        y1 = jnp.dot(x[c], a, preferred_element_type=jnp.float32)
        out_lr[0, tc, c] = jnp.dot(at, y1, preferred_element_type=jnp.float32)


def kernel(lr_data, hr_data, lr_doy, hr_doy, target_doy):
    B, Tlr, C, Hl, Wl = lr_data.shape
    _, Thr, _, Hh, Wh = hr_data.shape
    Tt = target_doy.shape[0]
    Hu, Wu = Hl * _UP, Wl * _UP

    def bounds(doy):
        idx = jax.vmap(
            lambda d: jnp.searchsorted(d, target_doy, side='left'))(doy)
        i1 = jnp.clip(idx, 1, doy.shape[1] - 1).astype(jnp.int32)
        return i1 - 1, i1

    lr_i0, lr_i1 = bounds(lr_doy)
    hr_i0, hr_i1 = bounds(hr_doy)

    a_np = _resize_mat(Hl, Hu)
    a = jnp.asarray(a_np)
    at = jnp.asarray(np.ascontiguousarray(a_np.T))

    grid_spec = pltpu.PrefetchScalarGridSpec(
        num_scalar_prefetch=7,
        grid=(B, Tt),
        in_specs=[
            pl.BlockSpec((1, 1, C, Hl, Wl),
                         lambda b, t, *s: (b, s[0][b, t], 0, 0, 0)),
            pl.BlockSpec((1, 1, C, Hl, Wl),
                         lambda b, t, *s: (b, s[1][b, t], 0, 0, 0)),
            pl.BlockSpec((1, 1, C, Hh, Wh),
                         lambda b, t, *s: (b, s[2][b, t], 0, 0, 0)),
            pl.BlockSpec((1, 1, C, Hh, Wh),
                         lambda b, t, *s: (b, s[3][b, t], 0, 0, 0)),
            pl.BlockSpec((Hl, Hu), lambda b, t, *s: (0, 0)),
            pl.BlockSpec((Hu, Hl), lambda b, t, *s: (0, 0)),
        ],
        out_specs=[
            pl.BlockSpec((1, _TC, C, Hu, Wu),
                         lambda b, t, *s: (b, t // _TC, 0, 0, 0)),
            pl.BlockSpec((1, _TC, C, Hh, Wh),
                         lambda b, t, *s: (b, t // _TC, 0, 0, 0)),
        ],
    )

    out_lr, out_hr = pl.pallas_call(
        _fusion_body,
        grid_spec=grid_spec,
        out_shape=[
            jax.ShapeDtypeStruct((B, Tt, C, Hu, Wu), jnp.float32),
            jax.ShapeDtypeStruct((B, Tt, C, Hh, Wh), jnp.float32),
        ],
        compiler_params=pltpu.CompilerParams(
            dimension_semantics=("arbitrary", "arbitrary")),
    )(lr_i0, lr_i1, hr_i0, hr_i1, lr_doy, hr_doy, target_doy,
      lr_data, lr_data, hr_data, hr_data, a, at)
    return out_lr, out_hr


# speedup vs baseline: 1.3360x; 1.3360x over previous
import numpy as np
import jax
import jax.numpy as jnp
from jax.experimental import pallas as pl
from jax.experimental.pallas import tpu as pltpu

_UP = 4
_TC = 5


def _resize_mat(n_in: int, n_out: int) -> np.ndarray:
    scale = n_out / n_in
    sample = (np.arange(n_out) + 0.5) / scale - 0.5
    dist = np.abs(sample[None, :] - np.arange(n_in)[:, None])
    w = np.maximum(0.0, 1.0 - dist)
    w = w / w.sum(axis=0, keepdims=True)
    return w.astype(np.float32)


def _fusion_body(slr0, shr0, lr_doy_s, hr_doy_s, tgt_s,
                 lr_all, hr_all, a_ref, at_ref, out_lr, out_hr):
    b = pl.program_id(0)
    chunk = pl.program_id(1)
    a = a_ref[...]
    at = at_ref[...]

    def lerp_weight(doy_s, i0, tval):
        d0 = doy_s[b, i0].astype(jnp.float32)
        d1 = doy_s[b, i0 + 1].astype(jnp.float32)
        denom = jnp.where(d1 - d0 == 0.0, 1.0, d1 - d0)
        return jnp.clip((tval - d0) / denom, 0.0, 1.0)

    for j in range(_TC):
        t = chunk * _TC + j
        tval = tgt_s[t].astype(jnp.float32)

        ih = shr0[b, t]
        w_hr = lerp_weight(hr_doy_s, ih, tval)
        out_hr[0, j] = hr_all[0, ih] * (1.0 - w_hr) + hr_all[0, ih + 1] * w_hr

        il = slr0[b, t]
        w_lr = lerp_weight(lr_doy_s, il, tval)
        x = lr_all[0, il] * (1.0 - w_lr) + lr_all[0, il + 1] * w_lr
        for c in range(x.shape[0]):
            y1 = jnp.dot(x[c], a, preferred_element_type=jnp.float32)
            out_lr[0, j, c] = jnp.dot(at, y1,
                                      preferred_element_type=jnp.float32)


def kernel(lr_data, hr_data, lr_doy, hr_doy, target_doy):
    B, Tlr, C, Hl, Wl = lr_data.shape
    _, Thr, _, Hh, Wh = hr_data.shape
    Tt = target_doy.shape[0]
    Hu, Wu = Hl * _UP, Wl * _UP

    def bounds(doy):
        idx = jax.vmap(
            lambda d: jnp.searchsorted(d, target_doy, side='left'))(doy)
        i1 = jnp.clip(idx, 1, doy.shape[1] - 1).astype(jnp.int32)
        return i1 - 1

    lr_i0 = bounds(lr_doy)
    hr_i0 = bounds(hr_doy)

    a_np = _resize_mat(Hl, Hu)
    a = jnp.asarray(a_np)
    at = jnp.asarray(np.ascontiguousarray(a_np.T))

    grid_spec = pltpu.PrefetchScalarGridSpec(
        num_scalar_prefetch=5,
        grid=(B, Tt // _TC),
        in_specs=[
            pl.BlockSpec((1, Tlr, C, Hl, Wl),
                         lambda b, t, *s: (b, 0, 0, 0, 0)),
            pl.BlockSpec((1, Thr, C, Hh, Wh),
                         lambda b, t, *s: (b, 0, 0, 0, 0)),
            pl.BlockSpec((Hl, Hu), lambda b, t, *s: (0, 0)),
            pl.BlockSpec((Hu, Hl), lambda b, t, *s: (0, 0)),
        ],
        out_specs=[
            pl.BlockSpec((1, _TC, C, Hu, Wu),
                         lambda b, t, *s: (b, t, 0, 0, 0)),
            pl.BlockSpec((1, _TC, C, Hh, Wh),
                         lambda b, t, *s: (b, t, 0, 0, 0)),
        ],
    )

    out_lr, out_hr = pl.pallas_call(
        _fusion_body,
        grid_spec=grid_spec,
        out_shape=[
            jax.ShapeDtypeStruct((B, Tt, C, Hu, Wu), jnp.float32),
            jax.ShapeDtypeStruct((B, Tt, C, Hh, Wh), jnp.float32),
        ],
        compiler_params=pltpu.CompilerParams(
            dimension_semantics=("arbitrary", "arbitrary")),
    )(lr_i0, hr_i0, lr_doy, hr_doy, target_doy,
      lr_data, hr_data, a, at)
    return out_lr, out_hr


# speedup vs baseline: 1.4912x; 1.1161x over previous
import numpy as np
import jax
import jax.numpy as jnp
from jax.experimental import pallas as pl
from jax.experimental.pallas import tpu as pltpu

_UP = 4


def _resize_mat(n_in: int, n_out: int) -> np.ndarray:
    scale = n_out / n_in
    sample = (np.arange(n_out) + 0.5) / scale - 0.5
    dist = np.abs(sample[None, :] - np.arange(n_in)[:, None])
    w = np.maximum(0.0, 1.0 - dist)
    w = w / w.sum(axis=0, keepdims=True)
    return w.astype(np.float32)


def _fusion_body(slr0, shr0, lr_doy_s, hr_doy_s, tgt_s,
                 lr_all, hr_all, a_ref, at_ref, out_lr, out_hr):
    b = pl.program_id(0)
    a = a_ref[...]
    at = at_ref[...]
    n_t = tgt_s.shape[0]

    def lerp_weight(doy_s, i0, tval):
        d0 = doy_s[b, i0].astype(jnp.float32)
        d1 = doy_s[b, i0 + 1].astype(jnp.float32)
        denom = jnp.where(d1 - d0 == 0.0, 1.0, d1 - d0)
        return jnp.clip((tval - d0) / denom, 0.0, 1.0)

    for t in range(n_t):
        tval = tgt_s[t].astype(jnp.float32)

        ih = shr0[b, t]
        w_hr = lerp_weight(hr_doy_s, ih, tval)
        out_hr[0, t, 0] = (hr_all[0, ih, 0] * (1.0 - w_hr)
                           + hr_all[0, ih + 1, 0] * w_hr)

        il = slr0[b, t]
        w_lr = lerp_weight(lr_doy_s, il, tval)
        x = (lr_all[0, il, 0] * (1.0 - w_lr)
             + lr_all[0, il + 1, 0] * w_lr)
        y1 = jnp.dot(x, a, preferred_element_type=jnp.float32)
        out_lr[0, t, 0] = jnp.dot(at, y1, preferred_element_type=jnp.float32)


def kernel(lr_data, hr_data, lr_doy, hr_doy, target_doy):
    B, Tlr, C, Hl, Wl = lr_data.shape
    _, Thr, _, Hh, Wh = hr_data.shape
    Tt = target_doy.shape[0]
    Hu, Wu = Hl * _UP, Wl * _UP

    def bounds(doy):
        idx = jax.vmap(
            lambda d: jnp.searchsorted(d, target_doy, side='left'))(doy)
        i1 = jnp.clip(idx, 1, doy.shape[1] - 1).astype(jnp.int32)
        return i1 - 1

    lr_i0 = bounds(lr_doy)
    hr_i0 = bounds(hr_doy)

    a_np = _resize_mat(Hl, Hu)
    a = jnp.asarray(a_np)
    at = jnp.asarray(np.ascontiguousarray(a_np.T))

    grid_spec = pltpu.PrefetchScalarGridSpec(
        num_scalar_prefetch=5,
        grid=(B, C),
        in_specs=[
            pl.BlockSpec((1, Tlr, 1, Hl, Wl),
                         lambda b, c, *s: (b, 0, c, 0, 0)),
            pl.BlockSpec((1, Thr, 1, Hh, Wh),
                         lambda b, c, *s: (b, 0, c, 0, 0)),
            pl.BlockSpec((Hl, Hu), lambda b, c, *s: (0, 0)),
            pl.BlockSpec((Hu, Hl), lambda b, c, *s: (0, 0)),
        ],
        out_specs=[
            pl.BlockSpec((1, Tt, 1, Hu, Wu),
                         lambda b, c, *s: (b, 0, c, 0, 0)),
            pl.BlockSpec((1, Tt, 1, Hh, Wh),
                         lambda b, c, *s: (b, 0, c, 0, 0)),
        ],
    )

    out_lr, out_hr = pl.pallas_call(
        _fusion_body,
        grid_spec=grid_spec,
        out_shape=[
            jax.ShapeDtypeStruct((B, Tt, C, Hu, Wu), jnp.float32),
            jax.ShapeDtypeStruct((B, Tt, C, Hh, Wh), jnp.float32),
        ],
        compiler_params=pltpu.CompilerParams(
            dimension_semantics=("arbitrary", "arbitrary")),
    )(lr_i0, hr_i0, lr_doy, hr_doy, target_doy,
      lr_data, hr_data, a, at)
    return out_lr, out_hr


# speedup vs baseline: 1.5086x; 1.0117x over previous
import numpy as np
import jax
import jax.numpy as jnp
from jax.experimental import pallas as pl
from jax.experimental.pallas import tpu as pltpu

_UP = 4


def _resize_mat(n_in: int, n_out: int) -> np.ndarray:
    scale = n_out / n_in
    sample = (np.arange(n_out) + 0.5) / scale - 0.5
    dist = np.abs(sample[None, :] - np.arange(n_in)[:, None])
    w = np.maximum(0.0, 1.0 - dist)
    w = w / w.sum(axis=0, keepdims=True)
    return w.astype(np.float32)


_CC = 2


def _fusion_body(slr0, shr0, lr_doy_s, hr_doy_s, tgt_s,
                 lr_all, hr_all, a_ref, at_ref, out_lr, out_hr):
    b = pl.program_id(0)
    a = a_ref[...]
    at = at_ref[...]
    n_t = tgt_s.shape[0]

    def lerp_weight(doy_s, i0, tval):
        d0 = doy_s[b, i0].astype(jnp.float32)
        d1 = doy_s[b, i0 + 1].astype(jnp.float32)
        denom = jnp.where(d1 - d0 == 0.0, 1.0, d1 - d0)
        return jnp.clip((tval - d0) / denom, 0.0, 1.0)

    for t in range(n_t):
        tval = tgt_s[t].astype(jnp.float32)

        ih = shr0[b, t]
        w_hr = lerp_weight(hr_doy_s, ih, tval)
        out_hr[0, t] = (hr_all[0, ih] * (1.0 - w_hr)
                        + hr_all[0, ih + 1] * w_hr)

        il = slr0[b, t]
        w_lr = lerp_weight(lr_doy_s, il, tval)
        x = (lr_all[0, il] * (1.0 - w_lr)
             + lr_all[0, il + 1] * w_lr)
        for c in range(_CC):
            y1 = jnp.dot(x[c], a, preferred_element_type=jnp.float32)
            out_lr[0, t, c] = jnp.dot(at, y1,
                                      preferred_element_type=jnp.float32)


def kernel(lr_data, hr_data, lr_doy, hr_doy, target_doy):
    B, Tlr, C, Hl, Wl = lr_data.shape
    _, Thr, _, Hh, Wh = hr_data.shape
    Tt = target_doy.shape[0]
    Hu, Wu = Hl * _UP, Wl * _UP

    def bounds(doy):
        idx = jax.vmap(
            lambda d: jnp.searchsorted(d, target_doy, side='left'))(doy)
        i1 = jnp.clip(idx, 1, doy.shape[1] - 1).astype(jnp.int32)
        return i1 - 1

    lr_i0 = bounds(lr_doy)
    hr_i0 = bounds(hr_doy)

    a_np = _resize_mat(Hl, Hu)
    a = jnp.asarray(a_np)
    at = jnp.asarray(np.ascontiguousarray(a_np.T))

    grid_spec = pltpu.PrefetchScalarGridSpec(
        num_scalar_prefetch=5,
        grid=(B, C // _CC),
        in_specs=[
            pl.BlockSpec((1, Tlr, _CC, Hl, Wl),
                         lambda b, c, *s: (b, 0, c, 0, 0)),
            pl.BlockSpec((1, Thr, _CC, Hh, Wh),
                         lambda b, c, *s: (b, 0, c, 0, 0)),
            pl.BlockSpec((Hl, Hu), lambda b, c, *s: (0, 0)),
            pl.BlockSpec((Hu, Hl), lambda b, c, *s: (0, 0)),
        ],
        out_specs=[
            pl.BlockSpec((1, Tt, _CC, Hu, Wu),
                         lambda b, c, *s: (b, 0, c, 0, 0)),
            pl.BlockSpec((1, Tt, _CC, Hh, Wh),
                         lambda b, c, *s: (b, 0, c, 0, 0)),
        ],
    )

    out_lr, out_hr = pl.pallas_call(
        _fusion_body,
        grid_spec=grid_spec,
        out_shape=[
            jax.ShapeDtypeStruct((B, Tt, C, Hu, Wu), jnp.float32),
            jax.ShapeDtypeStruct((B, Tt, C, Hh, Wh), jnp.float32),
        ],
        compiler_params=pltpu.CompilerParams(
            dimension_semantics=("arbitrary", "arbitrary")),
    )(lr_i0, hr_i0, lr_doy, hr_doy, target_doy,
      lr_data, hr_data, a, at)
    return out_lr, out_hr


# speedup vs baseline: 1.5089x; 1.0002x over previous
import numpy as np
import jax
import jax.numpy as jnp
from jax.experimental import pallas as pl
from jax.experimental.pallas import tpu as pltpu

_UP = 4


def _resize_mat(n_in: int, n_out: int) -> np.ndarray:
    scale = n_out / n_in
    sample = (np.arange(n_out) + 0.5) / scale - 0.5
    dist = np.abs(sample[None, :] - np.arange(n_in)[:, None])
    w = np.maximum(0.0, 1.0 - dist)
    w = w / w.sum(axis=0, keepdims=True)
    return w.astype(np.float32)


_CC = 2


def _fusion_body(slr0, shr0, lr_doy_s, hr_doy_s, tgt_s,
                 lr_all, hr_all, a_ref, at_ref, out_lr, out_hr):
    b = pl.program_id(0)
    a = a_ref[...]
    at = at_ref[...]
    n_t = tgt_s.shape[0]

    def lerp_weight(doy_s, i0, tval):
        d0 = doy_s[b, i0].astype(jnp.float32)
        d1 = doy_s[b, i0 + 1].astype(jnp.float32)
        denom = jnp.where(d1 - d0 == 0.0, 1.0, d1 - d0)
        return jnp.clip((tval - d0) / denom, 0.0, 1.0)

    for t in range(n_t):
        tval = tgt_s[t].astype(jnp.float32)

        ih = shr0[b, t]
        w_hr = lerp_weight(hr_doy_s, ih, tval)
        out_hr[0, t] = (hr_all[0, ih] * (1.0 - w_hr)
                        + hr_all[0, ih + 1] * w_hr)

        il = slr0[b, t]
        w_lr = lerp_weight(lr_doy_s, il, tval)
        x = (lr_all[0, il] * (1.0 - w_lr)
             + lr_all[0, il + 1] * w_lr)
        for c in range(_CC):
            y1 = jnp.dot(x[c], a, preferred_element_type=jnp.float32)
            out_lr[0, t, c] = jnp.dot(at, y1,
                                      preferred_element_type=jnp.float32)


def kernel(lr_data, hr_data, lr_doy, hr_doy, target_doy):
    B, Tlr, C, Hl, Wl = lr_data.shape
    _, Thr, _, Hh, Wh = hr_data.shape
    Tt = target_doy.shape[0]
    Hu, Wu = Hl * _UP, Wl * _UP

    def bounds(doy):
        idx = jax.vmap(
            lambda d: jnp.searchsorted(d, target_doy, side='left'))(doy)
        i1 = jnp.clip(idx, 1, doy.shape[1] - 1).astype(jnp.int32)
        return i1 - 1

    lr_i0 = bounds(lr_doy)
    hr_i0 = bounds(hr_doy)

    a_np = _resize_mat(Hl, Hu)
    a = jnp.asarray(a_np)
    at = jnp.asarray(np.ascontiguousarray(a_np.T))

    grid_spec = pltpu.PrefetchScalarGridSpec(
        num_scalar_prefetch=5,
        grid=(B, C // _CC),
        in_specs=[
            pl.BlockSpec((1, Tlr, _CC, Hl, Wl),
                         lambda b, c, *s: (b, 0, c, 0, 0)),
            pl.BlockSpec((1, Thr, _CC, Hh, Wh),
                         lambda b, c, *s: (b, 0, c, 0, 0)),
            pl.BlockSpec((Hl, Hu), lambda b, c, *s: (0, 0)),
            pl.BlockSpec((Hu, Hl), lambda b, c, *s: (0, 0)),
        ],
        out_specs=[
            pl.BlockSpec((1, Tt, _CC, Hu, Wu),
                         lambda b, c, *s: (b, 0, c, 0, 0)),
            pl.BlockSpec((1, Tt, _CC, Hh, Wh),
                         lambda b, c, *s: (b, 0, c, 0, 0)),
        ],
    )

    out_lr, out_hr = pl.pallas_call(
        _fusion_body,
        grid_spec=grid_spec,
        out_shape=[
            jax.ShapeDtypeStruct((B, Tt, C, Hu, Wu), jnp.float32),
            jax.ShapeDtypeStruct((B, Tt, C, Hh, Wh), jnp.float32),
        ],
        compiler_params=pltpu.CompilerParams(
            dimension_semantics=("parallel", "arbitrary")),
    )(lr_i0, hr_i0, lr_doy, hr_doy, target_doy,
      lr_data, hr_data, a, at)
    return out_lr, out_hr


# speedup vs baseline: 1.5094x; 1.0004x over previous
import numpy as np
import jax
import jax.numpy as jnp
from jax.experimental import pallas as pl
from jax.experimental.pallas import tpu as pltpu

_UP = 4


def _resize_mat(n_in: int, n_out: int) -> np.ndarray:
    scale = n_out / n_in
    sample = (np.arange(n_out) + 0.5) / scale - 0.5
    dist = np.abs(sample[None, :] - np.arange(n_in)[:, None])
    w = np.maximum(0.0, 1.0 - dist)
    w = w / w.sum(axis=0, keepdims=True)
    return w.astype(np.float32)


_CC = 2


def _fusion_body(slr0, shr0, lr_doy_s, hr_doy_s, tgt_s,
                 lr_all, hr_all, a_ref, at_ref, out_lr, out_hr):
    b = pl.program_id(0)
    a = a_ref[...]
    at = at_ref[...]
    n_t = tgt_s.shape[0]

    def lerp_weight(doy_s, i0, tval):
        d0 = doy_s[b, i0].astype(jnp.float32)
        d1 = doy_s[b, i0 + 1].astype(jnp.float32)
        denom = jnp.where(d1 - d0 == 0.0, 1.0, d1 - d0)
        return jnp.clip((tval - d0) / denom, 0.0, 1.0)

    for t in range(n_t):
        tval = tgt_s[t].astype(jnp.float32)

        ih = shr0[b, t]
        w_hr = lerp_weight(hr_doy_s, ih, tval)
        out_hr[0, t] = (hr_all[0, ih] * (1.0 - w_hr)
                        + hr_all[0, ih + 1] * w_hr)

        il = slr0[b, t]
        w_lr = lerp_weight(lr_doy_s, il, tval)
        x = (lr_all[0, il] * (1.0 - w_lr)
             + lr_all[0, il + 1] * w_lr)
        for c in range(_CC):
            y1 = jnp.dot(x[c], a, preferred_element_type=jnp.float32)
            out_lr[0, t, c] = jnp.dot(at, y1,
                                      preferred_element_type=jnp.float32)


def kernel(lr_data, hr_data, lr_doy, hr_doy, target_doy):
    B, Tlr, C, Hl, Wl = lr_data.shape
    _, Thr, _, Hh, Wh = hr_data.shape
    Tt = target_doy.shape[0]
    Hu, Wu = Hl * _UP, Wl * _UP

    def bounds(doy):
        idx = jax.vmap(
            lambda d: jnp.searchsorted(d, target_doy, side='left'))(doy)
        i1 = jnp.clip(idx, 1, doy.shape[1] - 1).astype(jnp.int32)
        return i1 - 1

    lr_i0 = bounds(lr_doy)
    hr_i0 = bounds(hr_doy)

    a_np = _resize_mat(Hl, Hu)
    a = jnp.asarray(a_np)
    at = jnp.asarray(np.ascontiguousarray(a_np.T))

    grid_spec = pltpu.PrefetchScalarGridSpec(
        num_scalar_prefetch=5,
        grid=(B, C // _CC),
        in_specs=[
            pl.BlockSpec((1, Tlr, _CC, Hl, Wl),
                         lambda b, c, *s: (b, 0, c, 0, 0)),
            pl.BlockSpec((1, Thr, _CC, Hh, Wh),
                         lambda b, c, *s: (b, 0, c, 0, 0)),
            pl.BlockSpec((Hl, Hu), lambda b, c, *s: (0, 0)),
            pl.BlockSpec((Hu, Hl), lambda b, c, *s: (0, 0)),
        ],
        out_specs=[
            pl.BlockSpec((1, Tt, _CC, Hu, Wu),
                         lambda b, c, *s: (b, 0, c, 0, 0)),
            pl.BlockSpec((1, Tt, _CC, Hh, Wh),
                         lambda b, c, *s: (b, 0, c, 0, 0)),
        ],
    )

    out_lr, out_hr = pl.pallas_call(
        _fusion_body,
        grid_spec=grid_spec,
        out_shape=[
            jax.ShapeDtypeStruct((B, Tt, C, Hu, Wu), jnp.float32),
            jax.ShapeDtypeStruct((B, Tt, C, Hh, Wh), jnp.float32),
        ],
        compiler_params=pltpu.CompilerParams(
            dimension_semantics=("arbitrary", "arbitrary")),
    )(lr_i0, hr_i0, lr_doy, hr_doy, target_doy,
      lr_data, hr_data, a, at)
    return out_lr, out_hr
